# baseline (speedup 1.0000x reference)
import functools

import jax
import jax.numpy as jnp
from jax.experimental import pallas as pl
from jax.experimental.pallas import tpu as pltpu

G = 1


def _sortable_i32(x):
    i = jax.lax.bitcast_convert_type(x, jnp.int32)
    return jnp.where(i < 0, i ^ jnp.int32(0x7FFFFFFF), i)


def _mgcc_kernel(ks, N, D, x1_ref, x2_ref, g1_ref, b1_ref, wrep_ref,
                 brep_ref, g2_ref, b2_ref, aw_ref, out_ref):
    f32 = jnp.float32
    g1 = g1_ref[...]
    b1 = b1_ref[...]

    def ln(x, g, b):
        mu = jnp.mean(x, axis=-1, keepdims=True)
        var = jnp.mean(x * x, axis=-1, keepdims=True) - mu * mu
        return (x - mu) * jax.lax.rsqrt(var + 1e-5) * g + b

    qs_l, ctx_l, ikey_l, ec_l = [], [], [], []
    for i in range(G):
        x1 = x1_ref[pl.ds(i * N, N), :]
        x2 = x2_ref[pl.ds(i * N, N), :]
        n1 = ln(x1, g1, b1)
        n2 = ln(x2, g1, b1)

        km = jnp.max(n2, axis=0, keepdims=True)
        ke = jnp.exp(n2 - km)
        key_t = ke / jnp.sum(ke, axis=0, keepdims=True)
        qm = jnp.max(n2, axis=1, keepdims=True)
        qe = jnp.exp(n2 - qm)
        qs_l.append(qe / jnp.sum(qe, axis=1, keepdims=True))

        ctx = jax.lax.dot_general(
            key_t, n1, (((0,), (0,)), ((), ())),
            preferred_element_type=f32)
        ctx_l.append(ctx)
        ikey_l.append(_sortable_i32(ctx)[None])
        m = jnp.max(ctx, axis=1, keepdims=True)
        ec_l.append(jnp.exp(ctx - m))

    ikey3 = (jnp.concatenate(ikey_l, axis=0) if G > 1
             else ikey_l[0])
    kidx = jax.lax.broadcasted_iota(jnp.int32, (4 * G, 1, 1), 0) % 4
    kvec = jnp.where(kidx == 0, ks[0],
            jnp.where(kidx == 1, ks[1],
             jnp.where(kidx == 2, ks[2], ks[3]))).astype(jnp.int32)

    kvecf = kvec.astype(f32)
    ones_v = jnp.ones((D, 1), jnp.bfloat16)

    def count_ge(mask_src):
        m = mask_src.astype(jnp.bfloat16).reshape(4 * G * D, D)
        return jnp.dot(m, ones_v,
                       preferred_element_type=f32).reshape(4 * G, D, 1)

    ikey16 = (ikey3 >> 16).astype(jnp.int16)
    lo = jnp.full((4 * G, D, 1), jnp.int32(-2**15))
    hi = jnp.full((4 * G, D, 1), jnp.int32(2**15 - 1))
    for _ in range(16):
        mid = lo + ((hi - lo) >> 1)
        cnt = count_ge(ikey16 > mid.astype(jnp.int16))
        pred = cnt >= kvecf
        lo = jnp.where(pred, mid + 1, lo)
        hi = jnp.where(pred, hi, mid)

    lo = lo << 16
    hi = lo | jnp.int32(0xFFFF)
    for _ in range(16):
        mid = lo + ((hi - lo) >> 1)
        cnt = count_ge(ikey3 > mid)
        pred = cnt >= kvecf
        lo = jnp.where(pred, mid + 1, lo)
        hi = jnp.where(pred, hi, mid)
    thr = lo

    mask3 = (ikey3 >= thr).astype(f32)
    ec3 = (jnp.concatenate([e[None] for e in ec_l], axis=0) if G > 1
           else ec_l[0][None])
    S = jnp.sum(ec3 * mask3, axis=2, keepdims=True)
    aw3 = jnp.where(kidx == 0, aw_ref[0],
           jnp.where(kidx == 1, aw_ref[1],
            jnp.where(kidx == 2, aw_ref[2], aw_ref[3])))
    coef = aw3.astype(f32) / S
    wtm = mask3 * coef

    for i in range(G):
        wt = jnp.sum(wtm[4 * i:4 * i + 4], axis=0)
        attn = ec_l[i] * wt
        attended = jax.lax.dot_general(
            attn.astype(jnp.bfloat16), qs_l[i].astype(jnp.bfloat16),
            (((1,), (1,)), ((), ())),
            preferred_element_type=f32)
        rep = jnp.dot(wrep_ref[...].astype(jnp.bfloat16),
                      attended.astype(jnp.bfloat16),
                      preferred_element_type=f32) + brep_ref[...]
        mu = jnp.mean(rep, axis=0, keepdims=True)
        var = jnp.mean(rep * rep, axis=0, keepdims=True) - mu * mu
        out = ((rep - mu) * jax.lax.rsqrt(var + 1e-5) * g2_ref[...]
               + b2_ref[...])
        out_ref[pl.ds(i * 2 * D, 2 * D), :] = out


def kernel(x1, x2, ln1_g, ln1_b, W_rep, b_rep, ln2_g, ln2_b, attn_w):
    B_, H_, W_, C_ = x1.shape
    N = H_ * W_
    D = C_
    ks = (int(D * 1 / 2), int(D * 2 / 3), int(D * 3 / 4), int(D * 4 / 5))

    x1f = x1.reshape(B_ * N, C_)
    x2f = x2.reshape(B_ * N, C_)
    g1 = ln1_g.reshape(1, C_)
    b1 = ln1_b.reshape(1, C_)
    brep = b_rep.reshape(2 * D, 1)
    g2 = ln2_g.reshape(2 * D, 1)
    b2 = ln2_b.reshape(2 * D, 1)

    out = pl.pallas_call(
        functools.partial(_mgcc_kernel, ks, N, D),
        grid=(B_ // G,),
        in_specs=[
            pl.BlockSpec((G * N, C_), lambda b: (b, 0)),
            pl.BlockSpec((G * N, C_), lambda b: (b, 0)),
            pl.BlockSpec((1, C_), lambda b: (0, 0)),
            pl.BlockSpec((1, C_), lambda b: (0, 0)),
            pl.BlockSpec((2 * D, D), lambda b: (0, 0)),
            pl.BlockSpec((2 * D, 1), lambda b: (0, 0)),
            pl.BlockSpec((2 * D, 1), lambda b: (0, 0)),
            pl.BlockSpec((2 * D, 1), lambda b: (0, 0)),
            pl.BlockSpec(memory_space=pltpu.SMEM),
        ],
        out_specs=pl.BlockSpec((G * 2 * D, N), lambda b: (b, 0)),
        out_shape=jax.ShapeDtypeStruct((B_ * 2 * D, N), jnp.float32),
    )(x1f, x2f, g1, b1, W_rep, brep, g2, b2, attn_w)

    return out.reshape(B_, 2 * D, H_, W_)


# speedup vs baseline: 1.1033x; 1.1033x over previous
import functools

import jax
import jax.numpy as jnp
from jax.experimental import pallas as pl
from jax.experimental.pallas import tpu as pltpu

G = 1


def _sortable_i32(x):
    i = jax.lax.bitcast_convert_type(x, jnp.int32)
    return jnp.where(i < 0, i ^ jnp.int32(0x7FFFFFFF), i)


def _mgcc_kernel(ks, N, D, x1_ref, x2_ref, g1_ref, b1_ref, wrep_ref,
                 brep_ref, g2_ref, b2_ref, aw_ref, out_ref):
    f32 = jnp.float32
    g1 = g1_ref[...]
    b1 = b1_ref[...]

    def ln(x, g, b):
        mu = jnp.mean(x, axis=-1, keepdims=True)
        var = jnp.mean(x * x, axis=-1, keepdims=True) - mu * mu
        return (x - mu) * jax.lax.rsqrt(var + 1e-5) * g + b

    qs_l, ctx_l, ikey_l, ec_l = [], [], [], []
    for i in range(G):
        x1 = x1_ref[pl.ds(i * N, N), :]
        x2 = x2_ref[pl.ds(i * N, N), :]
        n1 = ln(x1, g1, b1)
        n2 = ln(x2, g1, b1)

        km = jnp.max(n2, axis=0, keepdims=True)
        ke = jnp.exp(n2 - km)
        key_t = ke / jnp.sum(ke, axis=0, keepdims=True)
        qm = jnp.max(n2, axis=1, keepdims=True)
        qe = jnp.exp(n2 - qm)
        qs_l.append(qe / jnp.sum(qe, axis=1, keepdims=True))

        ctx = jax.lax.dot_general(
            key_t, n1, (((0,), (0,)), ((), ())),
            preferred_element_type=f32)
        ctx_l.append(ctx)
        ikey_l.append(_sortable_i32(ctx)[None])
        m = jnp.max(ctx, axis=1, keepdims=True)
        ec_l.append(jnp.exp(ctx - m))

    ikey3 = (jnp.concatenate(ikey_l, axis=0) if G > 1
             else ikey_l[0])
    kidx = jax.lax.broadcasted_iota(jnp.int32, (4 * G, 1, 1), 0) % 4
    kvec = jnp.where(kidx == 0, ks[0],
            jnp.where(kidx == 1, ks[1],
             jnp.where(kidx == 2, ks[2], ks[3]))).astype(jnp.int32)

    lo = jnp.full((4 * G, D, 1), jnp.int32(-2**31))
    hi = jnp.full((4 * G, D, 1), jnp.int32(2**31 - 1))
    kvecf = kvec.astype(f32)
    ones_v = jnp.ones((D, 1), jnp.bfloat16)
    for _ in range(32):
        mid = (lo & hi) + ((lo ^ hi) >> 1)
        mask = (ikey3 > mid).astype(jnp.bfloat16).reshape(4 * G * D, D)
        cnt = jnp.dot(mask, ones_v,
                      preferred_element_type=f32).reshape(4 * G, D, 1)
        pred = cnt >= kvecf
        lo = jnp.where(pred, mid + 1, lo)
        hi = jnp.where(pred, hi, mid)
    thr = lo

    mask3 = (ikey3 >= thr).astype(f32)
    ec3 = (jnp.concatenate([e[None] for e in ec_l], axis=0) if G > 1
           else ec_l[0][None])
    S = jnp.sum(ec3 * mask3, axis=2, keepdims=True)
    aw3 = jnp.where(kidx == 0, aw_ref[0],
           jnp.where(kidx == 1, aw_ref[1],
            jnp.where(kidx == 2, aw_ref[2], aw_ref[3])))
    coef = aw3.astype(f32) / S
    wtm = mask3 * coef

    for i in range(G):
        wt = jnp.sum(wtm[4 * i:4 * i + 4], axis=0)
        attn = ec_l[i] * wt
        attended = jax.lax.dot_general(
            attn.astype(jnp.bfloat16), qs_l[i].astype(jnp.bfloat16),
            (((1,), (1,)), ((), ())),
            preferred_element_type=f32)
        rep = jnp.dot(wrep_ref[...].astype(jnp.bfloat16),
                      attended.astype(jnp.bfloat16),
                      preferred_element_type=f32) + brep_ref[...]
        mu = jnp.mean(rep, axis=0, keepdims=True)
        var = jnp.mean(rep * rep, axis=0, keepdims=True) - mu * mu
        out = ((rep - mu) * jax.lax.rsqrt(var + 1e-5) * g2_ref[...]
               + b2_ref[...])
        out_ref[pl.ds(i * 2 * D, 2 * D), :] = out


def kernel(x1, x2, ln1_g, ln1_b, W_rep, b_rep, ln2_g, ln2_b, attn_w):
    B_, H_, W_, C_ = x1.shape
    N = H_ * W_
    D = C_
    ks = (int(D * 1 / 2), int(D * 2 / 3), int(D * 3 / 4), int(D * 4 / 5))

    x1f = x1.reshape(B_ * N, C_)
    x2f = x2.reshape(B_ * N, C_)
    g1 = ln1_g.reshape(1, C_)
    b1 = ln1_b.reshape(1, C_)
    brep = b_rep.reshape(2 * D, 1)
    g2 = ln2_g.reshape(2 * D, 1)
    b2 = ln2_b.reshape(2 * D, 1)

    out = pl.pallas_call(
        functools.partial(_mgcc_kernel, ks, N, D),
        grid=(B_ // G,),
        in_specs=[
            pl.BlockSpec((G * N, C_), lambda b: (b, 0)),
            pl.BlockSpec((G * N, C_), lambda b: (b, 0)),
            pl.BlockSpec((1, C_), lambda b: (0, 0)),
            pl.BlockSpec((1, C_), lambda b: (0, 0)),
            pl.BlockSpec((2 * D, D), lambda b: (0, 0)),
            pl.BlockSpec((2 * D, 1), lambda b: (0, 0)),
            pl.BlockSpec((2 * D, 1), lambda b: (0, 0)),
            pl.BlockSpec((2 * D, 1), lambda b: (0, 0)),
            pl.BlockSpec(memory_space=pltpu.SMEM),
        ],
        out_specs=pl.BlockSpec((G * 2 * D, N), lambda b: (b, 0)),
        out_shape=jax.ShapeDtypeStruct((B_ * 2 * D, N), jnp.float32),
        compiler_params=pltpu.CompilerParams(
            dimension_semantics=("parallel",)),
    )(x1f, x2f, g1, b1, W_rep, brep, g2, b2, attn_w)

    return out.reshape(B_, 2 * D, H_, W_)


# speedup vs baseline: 1.1371x; 1.0306x over previous
import functools

import jax
import jax.numpy as jnp
from jax.experimental import pallas as pl
from jax.experimental.pallas import tpu as pltpu

G = 1


def _sortable_i32(x):
    i = jax.lax.bitcast_convert_type(x, jnp.int32)
    return jnp.where(i < 0, i ^ jnp.int32(0x7FFFFFFF), i)


def _mgcc_kernel(ks, N, D, x1_ref, x2_ref, g1_ref, b1_ref, wrep_ref,
                 brep_ref, g2_ref, b2_ref, aw_ref, out_ref):
    f32 = jnp.float32
    g1 = g1_ref[...]
    b1 = b1_ref[...]

    def ln(x, g, b):
        mu = jnp.mean(x, axis=-1, keepdims=True)
        var = jnp.mean(x * x, axis=-1, keepdims=True) - mu * mu
        return (x - mu) * jax.lax.rsqrt(var + 1e-5) * g + b

    qs_l, ctx_l, ikey_l, ec_l = [], [], [], []
    for i in range(G):
        x1 = x1_ref[pl.ds(i * N, N), :]
        x2 = x2_ref[pl.ds(i * N, N), :]
        n1 = ln(x1, g1, b1)
        n2 = ln(x2, g1, b1)

        km = jnp.max(n2, axis=0, keepdims=True)
        ke = jnp.exp(n2 - km)
        key_t = ke / jnp.sum(ke, axis=0, keepdims=True)
        qm = jnp.max(n2, axis=1, keepdims=True)
        qe = jnp.exp(n2 - qm)
        qs_l.append(qe / jnp.sum(qe, axis=1, keepdims=True))

        ctx = jax.lax.dot_general(
            key_t, n1, (((0,), (0,)), ((), ())),
            preferred_element_type=f32)
        ctx_l.append(ctx)
        ikey_l.append(_sortable_i32(ctx)[None])
        m = jnp.max(ctx, axis=1, keepdims=True)
        ec_l.append(jnp.exp(ctx - m))

    ikey3 = (jnp.concatenate(ikey_l, axis=0) if G > 1
             else ikey_l[0])
    kidx = jax.lax.broadcasted_iota(jnp.int32, (4 * G, 1, 1), 0) % 4
    kvec = jnp.where(kidx == 0, ks[0],
            jnp.where(kidx == 1, ks[1],
             jnp.where(kidx == 2, ks[2], ks[3]))).astype(jnp.int32)

    ones_v = jnp.ones((D, 1), jnp.bfloat16)
    ikey2 = ikey_l[0][0]
    los = [jnp.full((D, 1), jnp.int32(-2**31)) for _ in range(4)]
    his = [jnp.full((D, 1), jnp.int32(2**31 - 1)) for _ in range(4)]
    kfs = [jnp.float32(k) for k in ks]
    for _ in range(32):
        for j in range(4):
            lo, hi = los[j], his[j]
            mid = (lo & hi) + ((lo ^ hi) >> 1)
            mask = (ikey2 > mid).astype(jnp.bfloat16)
            cnt = jnp.dot(mask, ones_v, preferred_element_type=f32)
            pred = cnt >= kfs[j]
            los[j] = jnp.where(pred, mid + 1, lo)
            his[j] = jnp.where(pred, hi, mid)
    thr = jnp.concatenate([t[None] for t in los], axis=0)

    mask3 = (ikey3 >= thr).astype(f32)
    ec3 = (jnp.concatenate([e[None] for e in ec_l], axis=0) if G > 1
           else ec_l[0][None])
    S = jnp.sum(ec3 * mask3, axis=2, keepdims=True)
    aw3 = jnp.where(kidx == 0, aw_ref[0],
           jnp.where(kidx == 1, aw_ref[1],
            jnp.where(kidx == 2, aw_ref[2], aw_ref[3])))
    coef = aw3.astype(f32) / S
    wtm = mask3 * coef

    for i in range(G):
        wt = jnp.sum(wtm[4 * i:4 * i + 4], axis=0)
        attn = ec_l[i] * wt
        attended = jax.lax.dot_general(
            attn.astype(jnp.bfloat16), qs_l[i].astype(jnp.bfloat16),
            (((1,), (1,)), ((), ())),
            preferred_element_type=f32)
        rep = jnp.dot(wrep_ref[...].astype(jnp.bfloat16),
                      attended.astype(jnp.bfloat16),
                      preferred_element_type=f32) + brep_ref[...]
        mu = jnp.mean(rep, axis=0, keepdims=True)
        var = jnp.mean(rep * rep, axis=0, keepdims=True) - mu * mu
        out = ((rep - mu) * jax.lax.rsqrt(var + 1e-5) * g2_ref[...]
               + b2_ref[...])
        out_ref[pl.ds(i * 2 * D, 2 * D), :] = out


def kernel(x1, x2, ln1_g, ln1_b, W_rep, b_rep, ln2_g, ln2_b, attn_w):
    B_, H_, W_, C_ = x1.shape
    N = H_ * W_
    D = C_
    ks = (int(D * 1 / 2), int(D * 2 / 3), int(D * 3 / 4), int(D * 4 / 5))

    x1f = x1.reshape(B_ * N, C_)
    x2f = x2.reshape(B_ * N, C_)
    g1 = ln1_g.reshape(1, C_)
    b1 = ln1_b.reshape(1, C_)
    brep = b_rep.reshape(2 * D, 1)
    g2 = ln2_g.reshape(2 * D, 1)
    b2 = ln2_b.reshape(2 * D, 1)

    out = pl.pallas_call(
        functools.partial(_mgcc_kernel, ks, N, D),
        grid=(B_ // G,),
        in_specs=[
            pl.BlockSpec((G * N, C_), lambda b: (b, 0)),
            pl.BlockSpec((G * N, C_), lambda b: (b, 0)),
            pl.BlockSpec((1, C_), lambda b: (0, 0)),
            pl.BlockSpec((1, C_), lambda b: (0, 0)),
            pl.BlockSpec((2 * D, D), lambda b: (0, 0)),
            pl.BlockSpec((2 * D, 1), lambda b: (0, 0)),
            pl.BlockSpec((2 * D, 1), lambda b: (0, 0)),
            pl.BlockSpec((2 * D, 1), lambda b: (0, 0)),
            pl.BlockSpec(memory_space=pltpu.SMEM),
        ],
        out_specs=pl.BlockSpec((G * 2 * D, N), lambda b: (b, 0)),
        out_shape=jax.ShapeDtypeStruct((B_ * 2 * D, N), jnp.float32),
        compiler_params=pltpu.CompilerParams(
            dimension_semantics=("parallel",)),
    )(x1f, x2f, g1, b1, W_rep, brep, g2, b2, attn_w)

    return out.reshape(B_, 2 * D, H_, W_)


# speedup vs baseline: 1.5113x; 1.3291x over previous
import functools

import jax
import jax.numpy as jnp
from jax.experimental import pallas as pl
from jax.experimental.pallas import tpu as pltpu

G = 1


def _sortable_i32(x):
    i = jax.lax.bitcast_convert_type(x, jnp.int32)
    return jnp.where(i < 0, i ^ jnp.int32(0x7FFFFFFF), i)


def _mgcc_kernel(ks, N, D, x1_ref, x2_ref, g1_ref, b1_ref, wrep_ref,
                 brep_ref, g2_ref, b2_ref, aw_ref, out_ref):
    f32 = jnp.float32
    g1 = g1_ref[...]
    b1 = b1_ref[...]

    def ln(x, g, b):
        mu = jnp.mean(x, axis=-1, keepdims=True)
        var = jnp.mean(x * x, axis=-1, keepdims=True) - mu * mu
        return (x - mu) * jax.lax.rsqrt(var + 1e-5) * g + b

    qs_l, ctx_l, ikey_l, ec_l = [], [], [], []
    for i in range(G):
        x1 = x1_ref[...].reshape(N, D)
        x2 = x2_ref[...].reshape(N, D)
        n1 = ln(x1, g1, b1)
        n2 = ln(x2, g1, b1)

        km = jnp.max(n2, axis=0, keepdims=True)
        ke = jnp.exp(n2 - km)
        key_t = ke / jnp.sum(ke, axis=0, keepdims=True)
        qm = jnp.max(n2, axis=1, keepdims=True)
        qe = jnp.exp(n2 - qm)
        qs_l.append(qe / jnp.sum(qe, axis=1, keepdims=True))

        ctx = jax.lax.dot_general(
            key_t, n1, (((0,), (0,)), ((), ())),
            preferred_element_type=f32)
        ctx_l.append(ctx)
        ikey_l.append(_sortable_i32(ctx)[None])
        m = jnp.max(ctx, axis=1, keepdims=True)
        ec_l.append(jnp.exp(ctx - m))

    ikey3 = (jnp.concatenate(ikey_l, axis=0) if G > 1
             else ikey_l[0])
    kidx = jax.lax.broadcasted_iota(jnp.int32, (4 * G, 1, 1), 0) % 4
    kvec = jnp.where(kidx == 0, ks[0],
            jnp.where(kidx == 1, ks[1],
             jnp.where(kidx == 2, ks[2], ks[3]))).astype(jnp.int32)

    ones_v = jnp.ones((D, 1), jnp.bfloat16)
    ikey2 = ikey_l[0][0]
    los = [jnp.full((D, 1), jnp.int32(-2**31)) for _ in range(4)]
    his = [jnp.full((D, 1), jnp.int32(2**31 - 1)) for _ in range(4)]
    kfs = [jnp.float32(k) for k in ks]
    for _ in range(32):
        for j in range(4):
            lo, hi = los[j], his[j]
            mid = (lo & hi) + ((lo ^ hi) >> 1)
            mask = (ikey2 > mid).astype(jnp.bfloat16)
            cnt = jnp.dot(mask, ones_v, preferred_element_type=f32)
            pred = cnt >= kfs[j]
            los[j] = jnp.where(pred, mid + 1, lo)
            his[j] = jnp.where(pred, hi, mid)
    thr = jnp.concatenate([t[None] for t in los], axis=0)

    mask3 = (ikey3 >= thr).astype(f32)
    ec3 = (jnp.concatenate([e[None] for e in ec_l], axis=0) if G > 1
           else ec_l[0][None])
    S = jnp.sum(ec3 * mask3, axis=2, keepdims=True)
    aw3 = jnp.where(kidx == 0, aw_ref[0],
           jnp.where(kidx == 1, aw_ref[1],
            jnp.where(kidx == 2, aw_ref[2], aw_ref[3])))
    coef = aw3.astype(f32) / S
    wtm = mask3 * coef

    for i in range(G):
        wt = jnp.sum(wtm[4 * i:4 * i + 4], axis=0)
        attn = ec_l[i] * wt
        attended = jax.lax.dot_general(
            attn.astype(jnp.bfloat16), qs_l[i].astype(jnp.bfloat16),
            (((1,), (1,)), ((), ())),
            preferred_element_type=f32)
        rep = jnp.dot(wrep_ref[...].astype(jnp.bfloat16),
                      attended.astype(jnp.bfloat16),
                      preferred_element_type=f32) + brep_ref[...]
        mu = jnp.mean(rep, axis=0, keepdims=True)
        var = jnp.mean(rep * rep, axis=0, keepdims=True) - mu * mu
        out = ((rep - mu) * jax.lax.rsqrt(var + 1e-5) * g2_ref[...]
               + b2_ref[...])
        out_ref[...] = out.reshape(1, 2 * D, N)


def kernel(x1, x2, ln1_g, ln1_b, W_rep, b_rep, ln2_g, ln2_b, attn_w):
    B_, H_, W_, C_ = x1.shape
    N = H_ * W_
    D = C_
    ks = (int(D * 1 / 2), int(D * 2 / 3), int(D * 3 / 4), int(D * 4 / 5))

    g1 = ln1_g.reshape(1, C_)
    b1 = ln1_b.reshape(1, C_)
    brep = b_rep.reshape(2 * D, 1)
    g2 = ln2_g.reshape(2 * D, 1)
    b2 = ln2_b.reshape(2 * D, 1)

    out = pl.pallas_call(
        functools.partial(_mgcc_kernel, ks, N, D),
        grid=(B_ // G,),
        in_specs=[
            pl.BlockSpec((1, H_, W_, C_), lambda b: (b, 0, 0, 0)),
            pl.BlockSpec((1, H_, W_, C_), lambda b: (b, 0, 0, 0)),
            pl.BlockSpec((1, C_), lambda b: (0, 0)),
            pl.BlockSpec((1, C_), lambda b: (0, 0)),
            pl.BlockSpec((2 * D, D), lambda b: (0, 0)),
            pl.BlockSpec((2 * D, 1), lambda b: (0, 0)),
            pl.BlockSpec((2 * D, 1), lambda b: (0, 0)),
            pl.BlockSpec((2 * D, 1), lambda b: (0, 0)),
            pl.BlockSpec(memory_space=pltpu.SMEM),
        ],
        out_specs=pl.BlockSpec((1, 2 * D, N), lambda b: (b, 0, 0)),
        out_shape=jax.ShapeDtypeStruct((B_, 2 * D, N), jnp.float32),
        compiler_params=pltpu.CompilerParams(
            dimension_semantics=("parallel",)),
    )(x1, x2, g1, b1, W_rep, brep, g2, b2, attn_w)

    return out.reshape(B_, 2 * D, H_, W_)
